# baseline (speedup 1.0000x reference)
import functools

import jax
import jax.numpy as jnp
import numpy as np
from jax import lax
from jax.experimental import pallas as pl
from jax.experimental.pallas import tpu as pltpu
from jax.experimental.pallas import tpu_sc as plsc

B = 4096
S = 200
D = 128
DW = D // 2
H = 512
VOUT = 1000

_NC = 2
_NS = 16
NW = _NC * _NS
BPW = B // NW
PAIRS = BPW // 2

_SPLITS = ((0, 104), (104, 104), (208, 104), (312, 88))

_SIG = np.empty(D, dtype=np.int32)
for _p in range(D):
    _k, _w = _p // 16, _p % 16
    _SIG[_p] = 16 * (_k // 2) + 64 * (_k % 2) + _w

V = 100000
_BV = 400


def _pack_body(a_ref, b_ref, o_ref):
    def rnd(x):
        return (x + 0x7FFF + ((x >> 16) & 1)) & jnp.uint32(0xFFFF0000)

    a = a_ref[...]
    b = b_ref[...]
    wa = rnd(a[:, DW:]) | (rnd(a[:, :DW]) >> 16)
    wb = rnd(b[:, DW:]) | (rnd(b[:, :DW]) >> 16)
    o_ref[...] = lax.bitcast_convert_type(
        jnp.concatenate([wa, wb], axis=1), jnp.int32)


def _pack_table(embed):
    er = lax.bitcast_convert_type(embed, jnp.uint32).reshape(V // 2, 2 * D)
    pk = pl.pallas_call(
        _pack_body,
        grid=(V // 2 // _BV,),
        in_specs=[
            pl.BlockSpec((_BV, D), lambda i: (i, 0)),
            pl.BlockSpec((_BV, D), lambda i: (i, 1)),
        ],
        out_specs=pl.BlockSpec((_BV, D), lambda i: (i, 0)),
        out_shape=jax.ShapeDtypeStruct((V // 2, D), jnp.int32),
    )(er, er)
    return pk.reshape(V, DW)


def _pool_sc(texts, packed):
    mesh = plsc.VectorSubcoreMesh(core_axis_name="c", subcore_axis_name="s")

    @functools.partial(
        pl.kernel,
        out_type=jax.ShapeDtypeStruct((B * D,), jnp.float32),
        mesh=mesh,
        compiler_params=pltpu.CompilerParams(use_tc_tiling_on_sc=False),
        scratch_types=[
            pltpu.VMEM((BPW * S,), jnp.int32),
            pltpu.VMEM((2 * S, DW), jnp.int32),
            pltpu.VMEM((2 * S, DW), jnp.int32),
            pltpu.VMEM((256,), jnp.float32),
            pltpu.SemaphoreType.DMA,
            pltpu.SemaphoreType.DMA,
            pltpu.SemaphoreType.DMA,
        ],
    )
    def k(texts_hbm, packed_hbm, out_hbm, idx_v, rows_a, rows_b, stage_v,
          sem_a, sem_b, sem_o):
        wid = lax.axis_index("s") * _NC + lax.axis_index("c")
        base = wid * BPW

        pltpu.sync_copy(texts_hbm.at[pl.ds(base * S, BPW * S)], idx_v)

        def fire_pair(p, rows_ref, sem):
            off = pl.multiple_of(p * 2 * S, 16)
            for o, n in _SPLITS:
                pltpu.async_copy(
                    packed_hbm.at[idx_v.at[pl.ds(off + o, n)]],
                    rows_ref.at[pl.ds(o, n)], sem)

        def wait_pair(rows_ref, sem):
            pltpu.make_async_copy(
                packed_hbm.at[pl.ds(0, 2 * S)], rows_ref, sem).wait()

        scale = jnp.float32(1.0 / S)
        himask = jnp.int32(-65536)

        def drain_out():
            pltpu.make_async_copy(
                out_hbm.at[pl.ds(0, D)], stage_v.at[pl.ds(0, D)],
                sem_o).wait()

        def accum_out(b, r0, rows_ref):
            def body(s, accs):
                new = list(accs)
                for j in range(4):
                    x = rows_ref[s, pl.ds(j * 16, 16)]
                    flo = lax.bitcast_convert_type(x << 16, jnp.float32)
                    fhi = lax.bitcast_convert_type(x & himask, jnp.float32)
                    new[2 * j] = new[2 * j] + flo
                    new[2 * j + 1] = new[2 * j + 1] + fhi
                return tuple(new)
            accs = lax.fori_loop(
                r0, r0 + S, body,
                tuple(jnp.zeros((16,), jnp.float32) for _ in range(8)))
            slot = (b % 2) * D

            @pl.when(b >= 2)
            def _():
                drain_out()

            for j in range(8):
                stage_v[pl.ds(slot + j * 16, 16)] = accs[j] * scale
            pltpu.async_copy(
                stage_v.at[pl.ds(slot, D)],
                out_hbm.at[pl.ds((base + b) * D, D)], sem_o)

        fire_pair(0, rows_a, sem_a)
        fire_pair(1, rows_b, sem_b)

        def loop_body(i, carry):
            p = i * 2
            b = p * 2
            wait_pair(rows_a, sem_a)
            accum_out(b, 0, rows_a)
            accum_out(b + 1, S, rows_a)

            @pl.when(p + 2 < PAIRS)
            def _():
                fire_pair(p + 2, rows_a, sem_a)

            wait_pair(rows_b, sem_b)
            accum_out(b + 2, 0, rows_b)
            accum_out(b + 3, S, rows_b)

            @pl.when(p + 3 < PAIRS)
            def _():
                fire_pair(p + 3, rows_b, sem_b)

            return carry

        lax.fori_loop(0, PAIRS // 2, loop_body, 0)
        drain_out()
        drain_out()

    return k(texts.reshape(B * S), packed)


def _mlp_body(p_ref, w1_ref, b1_ref, w2_ref, b2_ref, o_ref):
    h = jnp.dot(p_ref[...], w1_ref[...],
                preferred_element_type=jnp.float32) + b1_ref[...]
    h = jnp.maximum(h, 0.0)
    o_ref[...] = jnp.dot(h, w2_ref[...],
                         preferred_element_type=jnp.float32) + b2_ref[...]


def _mlp_tc(pooled, W1p, b1, W2, b2):
    BM = 512
    return pl.pallas_call(
        _mlp_body,
        grid=(B // BM,),
        in_specs=[
            pl.BlockSpec((BM, D), lambda i: (i, 0)),
            pl.BlockSpec((D, H), lambda i: (0, 0)),
            pl.BlockSpec((1, H), lambda i: (0, 0)),
            pl.BlockSpec((H, VOUT), lambda i: (0, 0)),
            pl.BlockSpec((1, VOUT), lambda i: (0, 0)),
        ],
        out_specs=pl.BlockSpec((BM, VOUT), lambda i: (i, 0)),
        out_shape=jax.ShapeDtypeStruct((B, VOUT), jnp.float32),
    )(pooled, W1p, b1.reshape(1, H), W2, b2.reshape(1, VOUT))


def kernel(texts, embed, W1, b1, W2, b2):
    packed = _pack_table(embed)
    pooled = _pool_sc(texts, packed).reshape(B, D)
    W1p = W1[jnp.asarray(_SIG), :]
    return _mlp_tc(pooled, W1p, b1, W2, b2)


# speedup vs baseline: 1.1953x; 1.1953x over previous
import functools

import jax
import jax.numpy as jnp
import numpy as np
from jax import lax
from jax.experimental import pallas as pl
from jax.experimental.pallas import tpu as pltpu
from jax.experimental.pallas import tpu_sc as plsc

B = 4096
S = 200
D = 128
DW = D // 2
H = 512
VOUT = 1000

_NC = 2
_NS = 16
NW = _NC * _NS
BPW = B // NW
PAIRS = BPW // 2

_SPLITS = ((0, 104), (104, 104), (208, 104), (312, 88))

_SIG = np.empty(D, dtype=np.int32)
for _p in range(D):
    _k, _w = _p // 16, _p % 16
    _SIG[_p] = 16 * (_k // 2) + 64 * (_k % 2) + _w

V = 100000
_BV = 400


def _pack_body(x_ref, o_ref):
    def rnd(x):
        return (x + 0x7FFF + ((x >> 16) & 1)) & jnp.uint32(0xFFFF0000)

    x = lax.bitcast_convert_type(x_ref[...], jnp.uint32)
    x3 = x.reshape(_BV, 2, D)
    a = x3[:, 0, :]
    b = x3[:, 1, :]
    wa = rnd(a[:, DW:]) | (rnd(a[:, :DW]) >> 16)
    wb = rnd(b[:, DW:]) | (rnd(b[:, :DW]) >> 16)
    o_ref[...] = lax.bitcast_convert_type(
        jnp.concatenate([wa, wb], axis=1), jnp.int32)


def _pack_table(embed):
    pk = pl.pallas_call(
        _pack_body,
        grid=(V // 2 // _BV,),
        in_specs=[pl.BlockSpec((2 * _BV, D), lambda i: (i, 0))],
        out_specs=pl.BlockSpec((_BV, D), lambda i: (i, 0)),
        out_shape=jax.ShapeDtypeStruct((V // 2, D), jnp.int32),
    )(embed)
    return pk.reshape(V, DW)


def _pool_sc(texts, packed):
    mesh = plsc.VectorSubcoreMesh(core_axis_name="c", subcore_axis_name="s")

    @functools.partial(
        pl.kernel,
        out_type=jax.ShapeDtypeStruct((B * D,), jnp.float32),
        mesh=mesh,
        compiler_params=pltpu.CompilerParams(use_tc_tiling_on_sc=False),
        scratch_types=[
            pltpu.VMEM((BPW * S,), jnp.int32),
            pltpu.VMEM((2 * S, DW), jnp.int32),
            pltpu.VMEM((2 * S, DW), jnp.int32),
            pltpu.VMEM((256,), jnp.float32),
            pltpu.SemaphoreType.DMA,
            pltpu.SemaphoreType.DMA,
            pltpu.SemaphoreType.DMA,
        ],
    )
    def k(texts_hbm, packed_hbm, out_hbm, idx_v, rows_a, rows_b, stage_v,
          sem_a, sem_b, sem_o):
        wid = lax.axis_index("s") * _NC + lax.axis_index("c")
        base = wid * BPW

        pltpu.sync_copy(texts_hbm.at[pl.ds(base * S, BPW * S)], idx_v)

        def fire_pair(p, rows_ref, sem):
            off = pl.multiple_of(p * 2 * S, 16)
            for o, n in _SPLITS:
                pltpu.async_copy(
                    packed_hbm.at[idx_v.at[pl.ds(off + o, n)]],
                    rows_ref.at[pl.ds(o, n)], sem)

        def wait_pair(rows_ref, sem):
            pltpu.make_async_copy(
                packed_hbm.at[pl.ds(0, 2 * S)], rows_ref, sem).wait()

        scale = jnp.float32(1.0 / S)
        himask = jnp.int32(-65536)

        def drain_out():
            pltpu.make_async_copy(
                out_hbm.at[pl.ds(0, D)], stage_v.at[pl.ds(0, D)],
                sem_o).wait()

        def accum_out(b, r0, rows_ref):
            def body(s, accs):
                new = list(accs)
                for j in range(4):
                    x = rows_ref[s, pl.ds(j * 16, 16)]
                    flo = lax.bitcast_convert_type(x << 16, jnp.float32)
                    fhi = lax.bitcast_convert_type(x & himask, jnp.float32)
                    new[2 * j] = new[2 * j] + flo
                    new[2 * j + 1] = new[2 * j + 1] + fhi
                return tuple(new)
            accs = lax.fori_loop(
                r0, r0 + S, body,
                tuple(jnp.zeros((16,), jnp.float32) for _ in range(8)))
            slot = (b % 2) * D

            @pl.when(b >= 2)
            def _():
                drain_out()

            for j in range(8):
                stage_v[pl.ds(slot + j * 16, 16)] = accs[j] * scale
            pltpu.async_copy(
                stage_v.at[pl.ds(slot, D)],
                out_hbm.at[pl.ds((base + b) * D, D)], sem_o)

        fire_pair(0, rows_a, sem_a)
        fire_pair(1, rows_b, sem_b)

        def loop_body(i, carry):
            p = i * 2
            b = p * 2
            wait_pair(rows_a, sem_a)
            accum_out(b, 0, rows_a)
            accum_out(b + 1, S, rows_a)

            @pl.when(p + 2 < PAIRS)
            def _():
                fire_pair(p + 2, rows_a, sem_a)

            wait_pair(rows_b, sem_b)
            accum_out(b + 2, 0, rows_b)
            accum_out(b + 3, S, rows_b)

            @pl.when(p + 3 < PAIRS)
            def _():
                fire_pair(p + 3, rows_b, sem_b)

            return carry

        lax.fori_loop(0, PAIRS // 2, loop_body, 0)
        drain_out()
        drain_out()

    return k(texts.reshape(B * S), packed)


def _mlp_body(p_ref, w1_ref, b1_ref, w2_ref, b2_ref, o_ref):
    h = jnp.dot(p_ref[...], w1_ref[...],
                preferred_element_type=jnp.float32) + b1_ref[...]
    h = jnp.maximum(h, 0.0)
    o_ref[...] = jnp.dot(h, w2_ref[...],
                         preferred_element_type=jnp.float32) + b2_ref[...]


def _mlp_tc(pooled, W1p, b1, W2, b2):
    BM = 512
    return pl.pallas_call(
        _mlp_body,
        grid=(B // BM,),
        in_specs=[
            pl.BlockSpec((BM, D), lambda i: (i, 0)),
            pl.BlockSpec((D, H), lambda i: (0, 0)),
            pl.BlockSpec((1, H), lambda i: (0, 0)),
            pl.BlockSpec((H, VOUT), lambda i: (0, 0)),
            pl.BlockSpec((1, VOUT), lambda i: (0, 0)),
        ],
        out_specs=pl.BlockSpec((BM, VOUT), lambda i: (i, 0)),
        out_shape=jax.ShapeDtypeStruct((B, VOUT), jnp.float32),
    )(pooled, W1p, b1.reshape(1, H), W2, b2.reshape(1, VOUT))


def kernel(texts, embed, W1, b1, W2, b2):
    packed = _pack_table(embed)
    pooled = _pool_sc(texts, packed).reshape(B, D)
    W1p = W1[jnp.asarray(_SIG), :]
    return _mlp_tc(pooled, W1p, b1, W2, b2)


# speedup vs baseline: 1.6871x; 1.4114x over previous
import functools

import jax
import jax.numpy as jnp
from jax import lax
from jax.experimental import pallas as pl
from jax.experimental.pallas import tpu as pltpu
from jax.experimental.pallas import tpu_sc as plsc

B = 4096
S = 200
D = 128
H = 512
VOUT = 1000

_NC = 2
_NS = 16
NW = _NC * _NS
BPW = B // NW

S0 = 104
S1 = S - S0


def _pool_sc(texts, embed):
    mesh = plsc.VectorSubcoreMesh(core_axis_name="c", subcore_axis_name="s")

    @functools.partial(
        pl.kernel,
        out_type=jax.ShapeDtypeStruct((B * D,), jnp.float32),
        mesh=mesh,
        scratch_types=[
            pltpu.VMEM((BPW * S,), jnp.int32),
            pltpu.VMEM((S, D), jnp.float32),
            pltpu.VMEM((S, D), jnp.float32),
            pltpu.VMEM((S, D), jnp.float32),
            pltpu.VMEM((S, D), jnp.float32),
            pltpu.VMEM((256,), jnp.float32),
            pltpu.SemaphoreType.DMA,
            pltpu.SemaphoreType.DMA,
            pltpu.SemaphoreType.DMA,
            pltpu.SemaphoreType.DMA,
            pltpu.SemaphoreType.DMA,
        ],
    )
    def k(texts_hbm, embed_hbm, out_hbm, idx_v, r0, r1, r2, r3, stage_v,
          s0, s1, s2, s3, sem_o):
        wid = lax.axis_index("s") * _NC + lax.axis_index("c")
        base = wid * BPW

        pltpu.sync_copy(texts_hbm.at[pl.ds(base * S, BPW * S)], idx_v)

        bufs = ((r0, s0), (r1, s1), (r2, s2), (r3, s3))

        def fire(b, rows_ref, sem):
            off = pl.multiple_of(b * S, 8)
            pltpu.async_copy(
                embed_hbm.at[idx_v.at[pl.ds(off, S0)]],
                rows_ref.at[pl.ds(0, S0)], sem)
            pltpu.async_copy(
                embed_hbm.at[idx_v.at[pl.ds(off + S0, S1)]],
                rows_ref.at[pl.ds(S0, S1)], sem)

        def wait(rows_ref, sem):
            pltpu.make_async_copy(
                embed_hbm.at[pl.ds(0, S)], rows_ref, sem).wait()

        scale = jnp.float32(1.0 / S)

        def drain_out():
            pltpu.make_async_copy(
                out_hbm.at[pl.ds(0, D)], stage_v.at[pl.ds(0, D)],
                sem_o).wait()

        def accum_out(b, rows_ref):
            def body(s, accs):
                return tuple(accs[j] + rows_ref[s, pl.ds(j * 16, 16)]
                             for j in range(8))
            accs = lax.fori_loop(
                0, S, body,
                tuple(jnp.zeros((16,), jnp.float32) for _ in range(8)))
            slot = (b % 2) * D

            @pl.when(b >= 2)
            def _():
                drain_out()

            for j in range(8):
                stage_v[pl.ds(slot + j * 16, 16)] = accs[j] * scale
            pltpu.async_copy(
                stage_v.at[pl.ds(slot, D)],
                out_hbm.at[pl.ds((base + b) * D, D)], sem_o)

        for e in range(3):
            fire(e, *bufs[e])

        def loop_body(i, carry):
            b = i * 4
            for u in range(4):
                rows_ref, sem = bufs[u]
                wait(rows_ref, sem)
                accum_out(b + u, rows_ref)
                nxt = b + u + 3

                @pl.when(nxt < BPW)
                def _(nxt=nxt, u=u):
                    fire(nxt, *bufs[(u + 3) % 4])

            return carry

        lax.fori_loop(0, BPW // 4, loop_body, 0)
        drain_out()
        drain_out()

    return k(texts.reshape(B * S), embed)


def _mlp_body(p_ref, w1_ref, b1_ref, w2_ref, b2_ref, o_ref):
    h = jnp.dot(p_ref[...], w1_ref[...],
                preferred_element_type=jnp.float32) + b1_ref[...]
    h = jnp.maximum(h, 0.0)
    o_ref[...] = jnp.dot(h, w2_ref[...],
                         preferred_element_type=jnp.float32) + b2_ref[...]


def _mlp_tc(pooled, W1, b1, W2, b2):
    BM = 512
    return pl.pallas_call(
        _mlp_body,
        grid=(B // BM,),
        in_specs=[
            pl.BlockSpec((BM, D), lambda i: (i, 0)),
            pl.BlockSpec((D, H), lambda i: (0, 0)),
            pl.BlockSpec((1, H), lambda i: (0, 0)),
            pl.BlockSpec((H, VOUT), lambda i: (0, 0)),
            pl.BlockSpec((1, VOUT), lambda i: (0, 0)),
        ],
        out_specs=pl.BlockSpec((BM, VOUT), lambda i: (i, 0)),
        out_shape=jax.ShapeDtypeStruct((B, VOUT), jnp.float32),
    )(pooled, W1, b1.reshape(1, H), W2, b2.reshape(1, VOUT))


def kernel(texts, embed, W1, b1, W2, b2):
    pooled = _pool_sc(texts, embed).reshape(B, D)
    return _mlp_tc(pooled, W1, b1, W2, b2)
